# baseline (speedup 1.0000x reference)
import jax
import jax.numpy as jnp
from jax import lax
from jax.experimental import pallas as pl
from jax.experimental.pallas import tpu as pltpu

N_USERS = 1024
N_POSTS = 5000
ROW_BLK = 256
USER_BLK = 256
_GRID_POSTS = (N_POSTS + ROW_BLK - 1) // ROW_BLK
_GRID_USERS = N_USERS // USER_BLK


def _dotbf(a, b):
    return jnp.dot(a.astype(jnp.bfloat16), b.astype(jnp.bfloat16),
                   preferred_element_type=jnp.float32)


def _leaky_relu(x):
    return jnp.maximum(x, 0.2 * x)


def _elu(x):
    return jnp.where(x > 0, x, jnp.exp(jnp.minimum(x, 0.0)) - 1.0)


def _encode_kernel(posts_ref, wp_ref, bp_ref, wg_ref, out_ref):
    p = _dotbf(posts_ref[...], wp_ref[...]) + bp_ref[...]
    out_ref[...] = _dotbf(p, wg_ref[...])


def _gat_rows(maskf, hw_blk, hw_full, a1_ref, a2_ref):
    s1 = _dotbf(hw_blk, a1_ref[...])
    s2 = lax.dot_general(a2_ref[...].astype(jnp.bfloat16),
                         hw_full.astype(jnp.bfloat16),
                         (((1,), (1,)), ((), ())),
                         preferred_element_type=jnp.float32)
    m = _leaky_relu(s1 + jnp.max(s2))
    u1 = s1 - m
    v1 = 0.2 * s1 - m
    s2b = 0.2 * s2
    ex = jnp.exp(jnp.maximum(u1 + s2, v1 + s2b)) * maskf
    l = jnp.sum(ex, axis=1, keepdims=True)
    alpha = ex / l
    return _elu(_dotbf(alpha, hw_full))


def _gat1_kernel(adj_ref, hw_blk_ref, hw_full_ref, a1_ref, a2_ref, wg1_ref,
                 hw1_ref, mask_ref):
    maskf = (adj_ref[...] != 0).astype(jnp.float32)
    p1 = _gat_rows(maskf, hw_blk_ref[...], hw_full_ref[...], a1_ref, a2_ref)
    hw1_ref[...] = _dotbf(p1, wg1_ref[...])
    mask_ref[...] = maskf.astype(jnp.bfloat16)


def _gat2_kernel(mask_ref, hw_blk_ref, hw_full_ref, a1_ref, a2_ref,
                 wp0_ref, bp0_ref, wp1_ref, bp1_ref,
                 p2_ref, label_ref):
    maskf = mask_ref[...].astype(jnp.float32)
    p2 = _gat_rows(maskf, hw_blk_ref[...], hw_full_ref[...], a1_ref, a2_ref)
    p2_ref[...] = p2
    t = jnp.maximum(_dotbf(p2, wp0_ref[...]) + bp0_ref[...], 0.0)
    label_ref[...] = _dotbf(t, wp1_ref[...]) + bp1_ref[...]


def _user_kernel(users_ref, up_ref, p2_ref, wu_ref, bu_ref,
                 wu0a_ref, wu0b_ref, bu0_ref, wu1_ref, bu1_ref, out_ref):
    up = up_ref[...]
    u = _dotbf(users_ref[...], wu_ref[...]) + bu_ref[...]
    denom = jnp.sum(up, axis=1, keepdims=True) + 1e-9
    agg = _dotbf(up, p2_ref[...]) / denom
    h = jnp.maximum(_dotbf(u, wu0a_ref[...]) + _dotbf(agg, wu0b_ref[...])
                    + bu0_ref[...], 0.0)
    out_ref[...] = _dotbf(h, wu1_ref[...]) + bu1_ref[...]


def _full(shape):
    return pl.BlockSpec(shape, lambda i: (0,) * len(shape))


def _rows(ncols, blk=ROW_BLK):
    return pl.BlockSpec((blk, ncols), lambda i: (i, 0))


_PARAMS = pltpu.CompilerParams(dimension_semantics=("arbitrary",))


@jax.jit
def kernel(users, posts, post_adjs, up_masking, W_user, b_user, W_post, b_post,
           W_gat0, a1_0, a2_0, W_gat1, a1_1, a2_1,
           Wp0, bp0, Wp1, bp1, Wu0, bu0, Wu1, bu1):
    f32 = jnp.float32
    D = W_gat0.shape[0]

    hw0 = pl.pallas_call(
        _encode_kernel,
        grid=(_GRID_POSTS,),
        in_specs=[_rows(posts.shape[1]), _full(W_post.shape), _full((1, D)),
                  _full(W_gat0.shape)],
        out_specs=_rows(D),
        out_shape=jax.ShapeDtypeStruct((N_POSTS, D), f32),
        compiler_params=_PARAMS,
    )(posts, W_post, b_post.reshape(1, D), W_gat0)

    hw1, maskbf = pl.pallas_call(
        _gat1_kernel,
        grid=(_GRID_POSTS,),
        in_specs=[_rows(N_POSTS), _rows(D), _full((N_POSTS, D)),
                  _full((D, 1)), _full((1, D)), _full((D, D))],
        out_specs=[_rows(D), _rows(N_POSTS)],
        out_shape=[jax.ShapeDtypeStruct((N_POSTS, D), f32),
                   jax.ShapeDtypeStruct((N_POSTS, N_POSTS), jnp.bfloat16)],
        compiler_params=_PARAMS,
    )(post_adjs, hw0, hw0, a1_0.reshape(D, 1), a2_0.reshape(1, D), W_gat1)

    p2, post_label = pl.pallas_call(
        _gat2_kernel,
        grid=(_GRID_POSTS,),
        in_specs=[_rows(N_POSTS), _rows(D), _full((N_POSTS, D)),
                  _full((D, 1)), _full((1, D)),
                  _full(Wp0.shape), _full((1, Wp0.shape[1])),
                  _full(Wp1.shape), _full((1, 1))],
        out_specs=[_rows(D), _rows(1)],
        out_shape=[jax.ShapeDtypeStruct((N_POSTS, D), f32),
                   jax.ShapeDtypeStruct((N_POSTS, 1), f32)],
        compiler_params=_PARAMS,
    )(maskbf, hw1, hw1, a1_1.reshape(D, 1), a2_1.reshape(1, D),
      Wp0, bp0.reshape(1, -1), Wp1, bp1.reshape(1, 1))

    d_ue = W_user.shape[1]
    user_label = pl.pallas_call(
        _user_kernel,
        grid=(_GRID_USERS,),
        in_specs=[_rows(users.shape[1], USER_BLK), _rows(N_POSTS, USER_BLK),
                  _full((N_POSTS, D)),
                  _full(W_user.shape), _full((1, d_ue)),
                  _full((d_ue, Wu0.shape[1])), _full((D, Wu0.shape[1])),
                  _full((1, Wu0.shape[1])), _full(Wu1.shape), _full((1, 1))],
        out_specs=_rows(1, USER_BLK),
        out_shape=jax.ShapeDtypeStruct((N_USERS, 1), f32),
        compiler_params=_PARAMS,
    )(users, up_masking, p2, W_user, b_user.reshape(1, -1),
      Wu0[:d_ue], Wu0[d_ue:], bu0.reshape(1, -1), Wu1, bu1.reshape(1, 1))

    return (user_label, post_label)


# speedup vs baseline: 1.0007x; 1.0007x over previous
import jax
import jax.numpy as jnp
from jax import lax
from jax.experimental import pallas as pl
from jax.experimental.pallas import tpu as pltpu

N_USERS = 1024
N_POSTS = 5000
ROW_BLK = 256
USER_BLK = 256
_GRID_POSTS = (N_POSTS + ROW_BLK - 1) // ROW_BLK
_GRID_USERS = N_USERS // USER_BLK


def _dotbf(a, b):
    return jnp.dot(a.astype(jnp.bfloat16), b.astype(jnp.bfloat16),
                   preferred_element_type=jnp.float32)


def _leaky_relu(x):
    return jnp.maximum(x, 0.2 * x)


def _elu(x):
    return jnp.where(x > 0, x, jnp.exp(jnp.minimum(x, 0.0)) - 1.0)


def _encode_kernel(posts_ref, wp_ref, bp_ref, wg_ref, out_ref):
    p = _dotbf(posts_ref[...], wp_ref[...]) + bp_ref[...]
    out_ref[...] = _dotbf(p, wg_ref[...])


def _gat_rows(maskf, hw_blk, hw_full, a1_ref, a2_ref):
    s1 = _dotbf(hw_blk, a1_ref[...])
    s2 = lax.dot_general(a2_ref[...].astype(jnp.bfloat16),
                         hw_full.astype(jnp.bfloat16),
                         (((1,), (1,)), ((), ())),
                         preferred_element_type=jnp.float32)
    m = _leaky_relu(s1 + jnp.max(s2))
    u1 = s1 - m
    v1 = 0.2 * s1 - m
    s2b = 0.2 * s2
    ex = jnp.exp(jnp.maximum(u1 + s2, v1 + s2b)) * maskf
    l = jnp.sum(ex, axis=1, keepdims=True)
    alpha = ex / l
    return _elu(_dotbf(alpha, hw_full))


def _gat1_kernel(adj_ref, hw_blk_ref, hw_full_ref, a1_ref, a2_ref, wg1_ref,
                 hw1_ref, mask_ref):
    maskf = (adj_ref[...] != 0).astype(jnp.float32)
    p1 = _gat_rows(maskf, hw_blk_ref[...], hw_full_ref[...], a1_ref, a2_ref)
    hw1_ref[...] = _dotbf(p1, wg1_ref[...])
    mask_ref[...] = maskf.astype(jnp.bfloat16)


def _gat2_kernel(mask_ref, hw_blk_ref, hw_full_ref, a1_ref, a2_ref,
                 wp0_ref, bp0_ref, wp1_ref, bp1_ref,
                 p2_ref, label_ref):
    maskf = mask_ref[...].astype(jnp.float32)
    p2 = _gat_rows(maskf, hw_blk_ref[...], hw_full_ref[...], a1_ref, a2_ref)
    p2_ref[...] = p2
    t = jnp.maximum(_dotbf(p2, wp0_ref[...]) + bp0_ref[...], 0.0)
    label_ref[...] = _dotbf(t, wp1_ref[...]) + bp1_ref[...]


def _user_kernel(users_ref, up_ref, p2_ref, wu_ref, bu_ref,
                 wu0a_ref, wu0b_ref, bu0_ref, wu1_ref, bu1_ref, out_ref):
    up = up_ref[...]
    u = _dotbf(users_ref[...], wu_ref[...]) + bu_ref[...]
    denom = jnp.sum(up, axis=1, keepdims=True) + 1e-9
    agg = _dotbf(up, p2_ref[...]) / denom
    h = jnp.maximum(_dotbf(u, wu0a_ref[...]) + _dotbf(agg, wu0b_ref[...])
                    + bu0_ref[...], 0.0)
    out_ref[...] = _dotbf(h, wu1_ref[...]) + bu1_ref[...]


def _full(shape):
    return pl.BlockSpec(shape, lambda i: (0,) * len(shape))


def _rows(ncols, blk=ROW_BLK):
    return pl.BlockSpec((blk, ncols), lambda i: (i, 0))


_PARAMS = pltpu.CompilerParams(dimension_semantics=("parallel",))


@jax.jit
def kernel(users, posts, post_adjs, up_masking, W_user, b_user, W_post, b_post,
           W_gat0, a1_0, a2_0, W_gat1, a1_1, a2_1,
           Wp0, bp0, Wp1, bp1, Wu0, bu0, Wu1, bu1):
    f32 = jnp.float32
    D = W_gat0.shape[0]

    hw0 = pl.pallas_call(
        _encode_kernel,
        grid=(_GRID_POSTS,),
        in_specs=[_rows(posts.shape[1]), _full(W_post.shape), _full((1, D)),
                  _full(W_gat0.shape)],
        out_specs=_rows(D),
        out_shape=jax.ShapeDtypeStruct((N_POSTS, D), f32),
        compiler_params=_PARAMS,
    )(posts, W_post, b_post.reshape(1, D), W_gat0)

    hw1, maskbf = pl.pallas_call(
        _gat1_kernel,
        grid=(_GRID_POSTS,),
        in_specs=[_rows(N_POSTS), _rows(D), _full((N_POSTS, D)),
                  _full((D, 1)), _full((1, D)), _full((D, D))],
        out_specs=[_rows(D), _rows(N_POSTS)],
        out_shape=[jax.ShapeDtypeStruct((N_POSTS, D), f32),
                   jax.ShapeDtypeStruct((N_POSTS, N_POSTS), jnp.bfloat16)],
        compiler_params=_PARAMS,
    )(post_adjs, hw0, hw0, a1_0.reshape(D, 1), a2_0.reshape(1, D), W_gat1)

    p2, post_label = pl.pallas_call(
        _gat2_kernel,
        grid=(_GRID_POSTS,),
        in_specs=[_rows(N_POSTS), _rows(D), _full((N_POSTS, D)),
                  _full((D, 1)), _full((1, D)),
                  _full(Wp0.shape), _full((1, Wp0.shape[1])),
                  _full(Wp1.shape), _full((1, 1))],
        out_specs=[_rows(D), _rows(1)],
        out_shape=[jax.ShapeDtypeStruct((N_POSTS, D), f32),
                   jax.ShapeDtypeStruct((N_POSTS, 1), f32)],
        compiler_params=_PARAMS,
    )(maskbf, hw1, hw1, a1_1.reshape(D, 1), a2_1.reshape(1, D),
      Wp0, bp0.reshape(1, -1), Wp1, bp1.reshape(1, 1))

    d_ue = W_user.shape[1]
    user_label = pl.pallas_call(
        _user_kernel,
        grid=(_GRID_USERS,),
        in_specs=[_rows(users.shape[1], USER_BLK), _rows(N_POSTS, USER_BLK),
                  _full((N_POSTS, D)),
                  _full(W_user.shape), _full((1, d_ue)),
                  _full((d_ue, Wu0.shape[1])), _full((D, Wu0.shape[1])),
                  _full((1, Wu0.shape[1])), _full(Wu1.shape), _full((1, 1))],
        out_specs=_rows(1, USER_BLK),
        out_shape=jax.ShapeDtypeStruct((N_USERS, 1), f32),
        compiler_params=_PARAMS,
    )(users, up_masking, p2, W_user, b_user.reshape(1, -1),
      Wu0[:d_ue], Wu0[d_ue:], bu0.reshape(1, -1), Wu1, bu1.reshape(1, 1))

    return (user_label, post_label)


# speedup vs baseline: 1.1284x; 1.1276x over previous
import jax
import jax.numpy as jnp
from jax import lax
from jax.experimental import pallas as pl
from jax.experimental.pallas import tpu as pltpu

N_USERS = 1024
N_POSTS = 5000
ROW_BLK = 512
USER_BLK = 256
_GRID_POSTS = (N_POSTS + ROW_BLK - 1) // ROW_BLK
_GRID_USERS = N_USERS // USER_BLK


def _dotbf(a, b):
    return jnp.dot(a.astype(jnp.bfloat16), b.astype(jnp.bfloat16),
                   preferred_element_type=jnp.float32)


def _leaky_relu(x):
    return jnp.maximum(x, 0.2 * x)


def _elu(x):
    return jnp.where(x > 0, x, jnp.exp(jnp.minimum(x, 0.0)) - 1.0)


def _encode_kernel(posts_ref, wp_ref, bp_ref, wg_ref, out_ref):
    p = _dotbf(posts_ref[...], wp_ref[...]) + bp_ref[...]
    out_ref[...] = _dotbf(p, wg_ref[...])


def _gat_rows(maskf, hw_blk, hw_full, a1_ref, a2_ref):
    s1 = _dotbf(hw_blk, a1_ref[...])
    s2 = lax.dot_general(a2_ref[...].astype(jnp.bfloat16),
                         hw_full.astype(jnp.bfloat16),
                         (((1,), (1,)), ((), ())),
                         preferred_element_type=jnp.float32)
    m = _leaky_relu(s1 + jnp.max(s2))
    u1 = s1 - m
    v1 = 0.2 * s1 - m
    s2b = 0.2 * s2
    ex = jnp.exp(jnp.maximum(u1 + s2, v1 + s2b)) * maskf
    l = jnp.sum(ex, axis=1, keepdims=True)
    alpha = ex / l
    return _elu(_dotbf(alpha, hw_full))


def _gat1_kernel(adj_ref, hw_blk_ref, hw_full_ref, a1_ref, a2_ref, wg1_ref,
                 hw1_ref, mask_ref):
    maskf = (adj_ref[...] != 0).astype(jnp.float32)
    p1 = _gat_rows(maskf, hw_blk_ref[...], hw_full_ref[...], a1_ref, a2_ref)
    hw1_ref[...] = _dotbf(p1, wg1_ref[...])
    mask_ref[...] = maskf.astype(jnp.bfloat16)


def _gat2_kernel(mask_ref, hw_blk_ref, hw_full_ref, a1_ref, a2_ref,
                 wp0_ref, bp0_ref, wp1_ref, bp1_ref,
                 p2_ref, label_ref):
    maskf = mask_ref[...].astype(jnp.float32)
    p2 = _gat_rows(maskf, hw_blk_ref[...], hw_full_ref[...], a1_ref, a2_ref)
    p2_ref[...] = p2
    t = jnp.maximum(_dotbf(p2, wp0_ref[...]) + bp0_ref[...], 0.0)
    label_ref[...] = _dotbf(t, wp1_ref[...]) + bp1_ref[...]


def _user_kernel(users_ref, up_ref, p2_ref, wu_ref, bu_ref,
                 wu0a_ref, wu0b_ref, bu0_ref, wu1_ref, bu1_ref, out_ref):
    up = up_ref[...]
    u = _dotbf(users_ref[...], wu_ref[...]) + bu_ref[...]
    denom = jnp.sum(up, axis=1, keepdims=True) + 1e-9
    agg = _dotbf(up, p2_ref[...]) / denom
    h = jnp.maximum(_dotbf(u, wu0a_ref[...]) + _dotbf(agg, wu0b_ref[...])
                    + bu0_ref[...], 0.0)
    out_ref[...] = _dotbf(h, wu1_ref[...]) + bu1_ref[...]


def _full(shape):
    return pl.BlockSpec(shape, lambda i: (0,) * len(shape))


def _rows(ncols, blk=ROW_BLK):
    return pl.BlockSpec((blk, ncols), lambda i: (i, 0))


_PARAMS = pltpu.CompilerParams(dimension_semantics=("parallel",))


@jax.jit
def kernel(users, posts, post_adjs, up_masking, W_user, b_user, W_post, b_post,
           W_gat0, a1_0, a2_0, W_gat1, a1_1, a2_1,
           Wp0, bp0, Wp1, bp1, Wu0, bu0, Wu1, bu1):
    f32 = jnp.float32
    D = W_gat0.shape[0]

    hw0 = pl.pallas_call(
        _encode_kernel,
        grid=(_GRID_POSTS,),
        in_specs=[_rows(posts.shape[1]), _full(W_post.shape), _full((1, D)),
                  _full(W_gat0.shape)],
        out_specs=_rows(D),
        out_shape=jax.ShapeDtypeStruct((N_POSTS, D), f32),
        compiler_params=_PARAMS,
    )(posts, W_post, b_post.reshape(1, D), W_gat0)

    hw1, maskbf = pl.pallas_call(
        _gat1_kernel,
        grid=(_GRID_POSTS,),
        in_specs=[_rows(N_POSTS), _rows(D), _full((N_POSTS, D)),
                  _full((D, 1)), _full((1, D)), _full((D, D))],
        out_specs=[_rows(D), _rows(N_POSTS)],
        out_shape=[jax.ShapeDtypeStruct((N_POSTS, D), f32),
                   jax.ShapeDtypeStruct((N_POSTS, N_POSTS), jnp.bfloat16)],
        compiler_params=_PARAMS,
    )(post_adjs, hw0, hw0, a1_0.reshape(D, 1), a2_0.reshape(1, D), W_gat1)

    p2, post_label = pl.pallas_call(
        _gat2_kernel,
        grid=(_GRID_POSTS,),
        in_specs=[_rows(N_POSTS), _rows(D), _full((N_POSTS, D)),
                  _full((D, 1)), _full((1, D)),
                  _full(Wp0.shape), _full((1, Wp0.shape[1])),
                  _full(Wp1.shape), _full((1, 1))],
        out_specs=[_rows(D), _rows(1)],
        out_shape=[jax.ShapeDtypeStruct((N_POSTS, D), f32),
                   jax.ShapeDtypeStruct((N_POSTS, 1), f32)],
        compiler_params=_PARAMS,
    )(maskbf, hw1, hw1, a1_1.reshape(D, 1), a2_1.reshape(1, D),
      Wp0, bp0.reshape(1, -1), Wp1, bp1.reshape(1, 1))

    d_ue = W_user.shape[1]
    user_label = pl.pallas_call(
        _user_kernel,
        grid=(_GRID_USERS,),
        in_specs=[_rows(users.shape[1], USER_BLK), _rows(N_POSTS, USER_BLK),
                  _full((N_POSTS, D)),
                  _full(W_user.shape), _full((1, d_ue)),
                  _full((d_ue, Wu0.shape[1])), _full((D, Wu0.shape[1])),
                  _full((1, Wu0.shape[1])), _full(Wu1.shape), _full((1, 1))],
        out_specs=_rows(1, USER_BLK),
        out_shape=jax.ShapeDtypeStruct((N_USERS, 1), f32),
        compiler_params=_PARAMS,
    )(users, up_masking, p2, W_user, b_user.reshape(1, -1),
      Wu0[:d_ue], Wu0[d_ue:], bu0.reshape(1, -1), Wu1, bu1.reshape(1, 1))

    return (user_label, post_label)
